# Initial kernel scaffold; baseline (speedup 1.0000x reference)
import jax
import jax.numpy as jnp
from jax.experimental import pallas as pl


def kernel(x_agent, x_track, edge_at, edge_ta, params):
    raise NotImplementedError("write your pallas kernel here")


# speedup vs baseline: 7.1475x; 7.1475x over previous
import functools

import jax
import jax.numpy as jnp
from jax import lax
from jax.experimental import pallas as pl
from jax.experimental.pallas import tpu as pltpu
from jax.experimental.pallas import tpu_sc as plsc

NA = 10000
NT = 10000
E = 320000
D = 128
H = 128

NSUB = 16
NPAD = 10240
RPT = NPAD // NSUB
K = 125
NCH = E // K
CPT = NCH // NSUB
IB = 16
NSTAGE = CPT // IB
BLK = 1000
NB_TYPE = NA // BLK


def _segsum_body(table, src, dst, zrow, zdeg, onesb,
                 out_sum, out_deg,
                 acc, dacc, src_v, dst_v, rows0, rows1, ones_v, sem0, sem1):
    c = lax.axis_index("c")
    s = lax.axis_index("s")
    pltpu.sync_copy(zrow, acc.at[pl.ds(s * RPT, RPT)])
    pltpu.sync_copy(zdeg, dacc.at[pl.ds(s * RPT, RPT)])
    pltpu.sync_copy(onesb, ones_v)
    plsc.subcore_barrier()
    rows = (rows0, rows1)
    sems = (sem0, sem1)

    def scatter(u):
        pltpu.sync_copy(rows[u % 2], acc.at[dst_v.at[u]], add=True)
        pltpu.sync_copy(ones_v, dacc.at[dst_v.at[u]], add=True)

    def stage(t, carry):
        base = s * CPT + t * IB
        pltpu.sync_copy(src.at[c, pl.ds(base, IB)], src_v)
        pltpu.sync_copy(dst.at[c, pl.ds(base, IB)], dst_v)
        cps = [None, None]
        for u in range(IB):
            cps[u % 2] = pltpu.async_copy(
                table.at[src_v.at[u]], rows[u % 2], sems[u % 2])
            if u > 0:
                cps[(u - 1) % 2].wait()
                scatter(u - 1)
        cps[(IB - 1) % 2].wait()
        scatter(IB - 1)
        return carry

    lax.fori_loop(0, NSTAGE, stage, 0)
    plsc.subcore_barrier()
    pltpu.sync_copy(acc.at[pl.ds(s * RPT, RPT)], out_sum.at[c, pl.ds(s * RPT, RPT)])
    pltpu.sync_copy(dacc.at[pl.ds(s * RPT, RPT)], out_deg.at[c, pl.ds(s * RPT, RPT)])


_segsum = pl.kernel(
    _segsum_body,
    out_type=(
        jax.ShapeDtypeStruct((2, NPAD, H), jnp.float32),
        jax.ShapeDtypeStruct((2, NPAD), jnp.float32),
    ),
    mesh=plsc.VectorSubcoreMesh(core_axis_name="c", subcore_axis_name="s"),
    scratch_types=[
        pltpu.VMEM_SHARED((NPAD, H), jnp.float32),
        pltpu.VMEM_SHARED((NPAD,), jnp.float32),
        pltpu.VMEM((IB, K), jnp.int32),
        pltpu.VMEM((IB, K), jnp.int32),
        pltpu.VMEM((K, H), jnp.float32),
        pltpu.VMEM((K, H), jnp.float32),
        pltpu.VMEM((K,), jnp.float32),
        pltpu.SemaphoreType.DMA,
        pltpu.SemaphoreType.DMA,
    ],
)


def _embed_body(x_ref, w_ref, b_ref, o_ref):
    y = jnp.dot(x_ref[...], w_ref[0], preferred_element_type=jnp.float32)
    o_ref[...] = jnp.maximum(y + b_ref[0], 0.0)


def _embed(xs, w, b):
    return pl.pallas_call(
        _embed_body,
        grid=(2 * NB_TYPE,),
        in_specs=[
            pl.BlockSpec((BLK, D), lambda i: (i, 0)),
            pl.BlockSpec((1, D, H), lambda i: (i // NB_TYPE, 0, 0)),
            pl.BlockSpec((1, 1, H), lambda i: (i // NB_TYPE, 0, 0)),
        ],
        out_specs=pl.BlockSpec((BLK, H), lambda i: (i, 0)),
        out_shape=jax.ShapeDtypeStruct((NA + NT, H), jnp.float32),
    )(xs, w, b)


def _conv_body(h_ref, agg_ref, deg_ref, ws_ref, wn_ref, b_ref, o_ref):
    d = jnp.maximum(deg_ref[0, :, 0], 1.0)
    a = agg_ref[0] / d[:, None]
    y = jnp.dot(h_ref[...], ws_ref[0], preferred_element_type=jnp.float32)
    y = y + jnp.dot(a, wn_ref[0], preferred_element_type=jnp.float32)
    o_ref[...] = jnp.maximum(y + b_ref[0], 0.0)


def _conv(hs, sums, degs, ws, wn, b):
    return pl.pallas_call(
        _conv_body,
        grid=(2 * NB_TYPE,),
        in_specs=[
            pl.BlockSpec((BLK, H), lambda i: (i, 0)),
            pl.BlockSpec((1, BLK, H), lambda i: (1 - i // NB_TYPE, i % NB_TYPE, 0)),
            pl.BlockSpec((1, BLK, 1), lambda i: (1 - i // NB_TYPE, i % NB_TYPE, 0)),
            pl.BlockSpec((1, H, H), lambda i: (i // NB_TYPE, 0, 0)),
            pl.BlockSpec((1, H, H), lambda i: (i // NB_TYPE, 0, 0)),
            pl.BlockSpec((1, 1, H), lambda i: (i // NB_TYPE, 0, 0)),
        ],
        out_specs=pl.BlockSpec((BLK, H), lambda i: (i, 0)),
        out_shape=jax.ShapeDtypeStruct((NA + NT, H), jnp.float32),
    )(hs, sums, degs, ws, wn, b)


def _heads_body(h_ref, ws_ref, bs_ref, wh_ref, bh_ref, o_ref):
    h2 = jnp.dot(h_ref[...], ws_ref[0], preferred_element_type=jnp.float32)
    h2 = jnp.maximum(h2 + bs_ref[0], 0.0)
    y = jnp.dot(h2, wh_ref[0], preferred_element_type=jnp.float32)
    y = y + bh_ref[0]
    col = lax.broadcasted_iota(jnp.int32, y.shape, 1)
    o_ref[...] = jnp.where(col < 2, jax.nn.sigmoid(y), y)


def _heads(hs, ws, bs, wh, bh):
    return pl.pallas_call(
        _heads_body,
        grid=(2 * NB_TYPE,),
        in_specs=[
            pl.BlockSpec((BLK, H), lambda i: (i, 0)),
            pl.BlockSpec((1, H, H), lambda i: (i // NB_TYPE, 0, 0)),
            pl.BlockSpec((1, 1, H), lambda i: (i // NB_TYPE, 0, 0)),
            pl.BlockSpec((1, H, 3), lambda i: (i // NB_TYPE, 0, 0)),
            pl.BlockSpec((1, 1, 3), lambda i: (i // NB_TYPE, 0, 0)),
        ],
        out_specs=pl.BlockSpec((BLK, 3), lambda i: (i, 0)),
        out_shape=jax.ShapeDtypeStruct((NA + NT, 3), jnp.float32),
    )(hs, ws, bs, wh, bh)


def kernel(x_agent, x_track, edge_at, edge_ta, params):
    p = params
    xs = jnp.concatenate([x_agent, x_track], axis=0)
    w_e = jnp.stack([p["W_ae"], p["W_te"]])
    b_e = jnp.stack([p["b_ae"], p["b_te"]])[:, None, :]

    src = jnp.stack([edge_at[0], edge_ta[0] + NA]).reshape(2, NCH, K)
    dst = jnp.stack([edge_at[1], edge_ta[1]]).reshape(2, NCH, K)
    zrow = jnp.zeros((RPT, H), jnp.float32)
    zdeg = jnp.zeros((RPT,), jnp.float32)
    onesb = jnp.ones((K,), jnp.float32)

    hs = _embed(xs, w_e, b_e)
    for conv in (p["conv1"], p["conv2"]):
        sums, degs = _segsum(hs, src, dst, zrow, zdeg, onesb)
        degs = degs[:, :, None]
        ws = jnp.stack([conv["Ws_a"], conv["Ws_t"]])
        wn = jnp.stack([conv["Wn_a"], conv["Wn_t"]])
        bb = jnp.stack([conv["b_a"], conv["b_t"]])[:, None, :]
        hs = _conv(hs, sums, degs, ws, wn, bb)

    w_sym = jnp.stack([p["W_sym_a"], p["W_sym_t"]])
    b_sym = jnp.stack([p["b_sym_a"], p["b_sym_t"]])[:, None, :]
    wh = jnp.stack([
        jnp.concatenate([p["W_apv"], p["W_apc"], p["W_av"]], axis=1),
        jnp.concatenate([p["W_tpv"], p["W_tpc"], p["W_tv"]], axis=1),
    ])
    bh = jnp.stack([
        jnp.concatenate([p["b_apv"], p["b_apc"], p["b_av"]]),
        jnp.concatenate([p["b_tpv"], p["b_tpc"], p["b_tv"]]),
    ])[:, None, :]
    out = _heads(hs, w_sym, b_sym, wh, bh)
    return (out[:NA, 0:1], out[:NA, 1:2], out[:NA, 2:3],
            out[NA:, 0:1], out[NA:, 1:2], out[NA:, 2:3])


# speedup vs baseline: 7.8369x; 1.0965x over previous
import functools

import jax
import jax.numpy as jnp
from jax import lax
from jax.experimental import pallas as pl
from jax.experimental.pallas import tpu as pltpu
from jax.experimental.pallas import tpu_sc as plsc

NA = 10000
NT = 10000
E = 320000
D = 128
H = 128

NSUB = 16
NPAD = 10240
RPT = NPAD // NSUB
K = 125
NCH = E // K
CPT = NCH // NSUB
IB = 32
NSTAGE = CPT // IB
BLK = 1000
NB_TYPE = NA // BLK


def _make_segsum(with_deg):
    def body(*refs):
        if with_deg:
            (table, src, dst, zrow, zdeg, onesb, out_sum, out_deg,
             acc, dacc, src_v, dst_v, rows0, rows1, ones_v,
             gsem0, gsem1, ssem0, ssem1, dsem) = refs
        else:
            (table, src, dst, zrow, out_sum,
             acc, src_v, dst_v, rows0, rows1,
             gsem0, gsem1, ssem0, ssem1) = refs
        c = lax.axis_index("c")
        s = lax.axis_index("s")
        pltpu.sync_copy(zrow, acc.at[pl.ds(s * RPT, RPT)])
        if with_deg:
            pltpu.sync_copy(zdeg, dacc.at[pl.ds(s * RPT, RPT)])
            pltpu.sync_copy(onesb, ones_v)
        plsc.subcore_barrier()
        rows = (rows0, rows1)
        gsems = (gsem0, gsem1)
        ssems = (ssem0, ssem1)

        def stage(t, carry):
            base = s * CPT + t * IB
            pltpu.sync_copy(src.at[c, pl.ds(base, IB)], src_v)
            pltpu.sync_copy(dst.at[c, pl.ds(base, IB)], dst_v)
            gd = [None, None]
            sd = [None, None]
            dd = [None] * IB
            gd[0] = pltpu.async_copy(table.at[src_v.at[0]], rows[0], gsems[0])
            for u in range(IB):
                if u + 1 < IB:
                    if sd[(u + 1) % 2] is not None:
                        sd[(u + 1) % 2].wait()
                    gd[(u + 1) % 2] = pltpu.async_copy(
                        table.at[src_v.at[u + 1]], rows[(u + 1) % 2],
                        gsems[(u + 1) % 2])
                gd[u % 2].wait()
                sd[u % 2] = pltpu.async_copy(
                    rows[u % 2], acc.at[dst_v.at[u]], ssems[u % 2], add=True)
                if with_deg:
                    dd[u] = pltpu.async_copy(
                        ones_v, dacc.at[dst_v.at[u]], dsem, add=True)
                    if u > 0:
                        dd[u - 1].wait()
            sd[0].wait()
            sd[1].wait()
            if with_deg:
                dd[IB - 1].wait()
            return carry

        lax.fori_loop(0, NSTAGE, stage, 0)
        plsc.subcore_barrier()
        pltpu.sync_copy(acc.at[pl.ds(s * RPT, RPT)],
                        out_sum.at[c, pl.ds(s * RPT, RPT)])
        if with_deg:
            pltpu.sync_copy(dacc.at[pl.ds(s * RPT, RPT)],
                            out_deg.at[c, pl.ds(s * RPT, RPT)])

    if with_deg:
        out_type = (
            jax.ShapeDtypeStruct((2, NPAD, H), jnp.float32),
            jax.ShapeDtypeStruct((2, NPAD), jnp.float32),
        )
    else:
        out_type = jax.ShapeDtypeStruct((2, NPAD, H), jnp.float32)
    scratch = [pltpu.VMEM_SHARED((NPAD, H), jnp.float32)]
    if with_deg:
        scratch.append(pltpu.VMEM_SHARED((NPAD,), jnp.float32))
    scratch += [
        pltpu.VMEM((IB, K), jnp.int32),
        pltpu.VMEM((IB, K), jnp.int32),
        pltpu.VMEM((K, H), jnp.float32),
        pltpu.VMEM((K, H), jnp.float32),
    ]
    if with_deg:
        scratch.append(pltpu.VMEM((K,), jnp.float32))
    scratch += [
        pltpu.SemaphoreType.DMA,
        pltpu.SemaphoreType.DMA,
        pltpu.SemaphoreType.DMA,
        pltpu.SemaphoreType.DMA,
    ]
    if with_deg:
        scratch.append(pltpu.SemaphoreType.DMA)
    return pl.kernel(
        body,
        out_type=out_type,
        mesh=plsc.VectorSubcoreMesh(core_axis_name="c", subcore_axis_name="s"),
        scratch_types=scratch,
    )


_segsum_deg = _make_segsum(True)
_segsum_nodeg = _make_segsum(False)


def _embed_body(x_ref, w_ref, b_ref, o_ref):
    y = jnp.dot(x_ref[...], w_ref[0], preferred_element_type=jnp.float32)
    o_ref[...] = jnp.maximum(y + b_ref[0], 0.0)


def _embed(xs, w, b):
    return pl.pallas_call(
        _embed_body,
        grid=(2 * NB_TYPE,),
        in_specs=[
            pl.BlockSpec((BLK, D), lambda i: (i, 0)),
            pl.BlockSpec((1, D, H), lambda i: (i // NB_TYPE, 0, 0)),
            pl.BlockSpec((1, 1, H), lambda i: (i // NB_TYPE, 0, 0)),
        ],
        out_specs=pl.BlockSpec((BLK, H), lambda i: (i, 0)),
        out_shape=jax.ShapeDtypeStruct((NA + NT, H), jnp.float32),
    )(xs, w, b)


def _conv_body(h_ref, agg_ref, deg_ref, ws_ref, wn_ref, b_ref, o_ref):
    d = jnp.maximum(deg_ref[0, :, 0], 1.0)
    a = agg_ref[0] / d[:, None]
    y = jnp.dot(h_ref[...], ws_ref[0], preferred_element_type=jnp.float32)
    y = y + jnp.dot(a, wn_ref[0], preferred_element_type=jnp.float32)
    o_ref[...] = jnp.maximum(y + b_ref[0], 0.0)


def _conv(hs, sums, degs, ws, wn, b):
    return pl.pallas_call(
        _conv_body,
        grid=(2 * NB_TYPE,),
        in_specs=[
            pl.BlockSpec((BLK, H), lambda i: (i, 0)),
            pl.BlockSpec((1, BLK, H), lambda i: (1 - i // NB_TYPE, i % NB_TYPE, 0)),
            pl.BlockSpec((1, BLK, 1), lambda i: (1 - i // NB_TYPE, i % NB_TYPE, 0)),
            pl.BlockSpec((1, H, H), lambda i: (i // NB_TYPE, 0, 0)),
            pl.BlockSpec((1, H, H), lambda i: (i // NB_TYPE, 0, 0)),
            pl.BlockSpec((1, 1, H), lambda i: (i // NB_TYPE, 0, 0)),
        ],
        out_specs=pl.BlockSpec((BLK, H), lambda i: (i, 0)),
        out_shape=jax.ShapeDtypeStruct((NA + NT, H), jnp.float32),
    )(hs, sums, degs, ws, wn, b)


def _conv_heads_body(h_ref, agg_ref, deg_ref, ws_ref, wn_ref, b_ref,
                     wsym_ref, bsym_ref, wh_ref, bh_ref, o_ref):
    d = jnp.maximum(deg_ref[0, :, 0], 1.0)
    a = agg_ref[0] / d[:, None]
    y = jnp.dot(h_ref[...], ws_ref[0], preferred_element_type=jnp.float32)
    y = y + jnp.dot(a, wn_ref[0], preferred_element_type=jnp.float32)
    y = jnp.maximum(y + b_ref[0], 0.0)
    h2 = jnp.dot(y, wsym_ref[0], preferred_element_type=jnp.float32)
    h2 = jnp.maximum(h2 + bsym_ref[0], 0.0)
    z = jnp.dot(h2, wh_ref[0], preferred_element_type=jnp.float32)
    z = z + bh_ref[0]
    col = lax.broadcasted_iota(jnp.int32, z.shape, 1)
    o_ref[...] = jnp.where(col < 2, jax.nn.sigmoid(z), z)


def _conv_heads(hs, sums, degs, ws, wn, b, wsym, bsym, wh, bh):
    return pl.pallas_call(
        _conv_heads_body,
        grid=(2 * NB_TYPE,),
        in_specs=[
            pl.BlockSpec((BLK, H), lambda i: (i, 0)),
            pl.BlockSpec((1, BLK, H), lambda i: (1 - i // NB_TYPE, i % NB_TYPE, 0)),
            pl.BlockSpec((1, BLK, 1), lambda i: (1 - i // NB_TYPE, i % NB_TYPE, 0)),
            pl.BlockSpec((1, H, H), lambda i: (i // NB_TYPE, 0, 0)),
            pl.BlockSpec((1, H, H), lambda i: (i // NB_TYPE, 0, 0)),
            pl.BlockSpec((1, 1, H), lambda i: (i // NB_TYPE, 0, 0)),
            pl.BlockSpec((1, H, H), lambda i: (i // NB_TYPE, 0, 0)),
            pl.BlockSpec((1, 1, H), lambda i: (i // NB_TYPE, 0, 0)),
            pl.BlockSpec((1, H, 3), lambda i: (i // NB_TYPE, 0, 0)),
            pl.BlockSpec((1, 1, 3), lambda i: (i // NB_TYPE, 0, 0)),
        ],
        out_specs=pl.BlockSpec((BLK, 3), lambda i: (i, 0)),
        out_shape=jax.ShapeDtypeStruct((NA + NT, 3), jnp.float32),
    )(hs, sums, degs, ws, wn, b, wsym, bsym, wh, bh)


def kernel(x_agent, x_track, edge_at, edge_ta, params):
    p = params
    xs = jnp.concatenate([x_agent, x_track], axis=0)
    w_e = jnp.stack([p["W_ae"], p["W_te"]])
    b_e = jnp.stack([p["b_ae"], p["b_te"]])[:, None, :]

    src = jnp.stack([edge_at[0], edge_ta[0] + NA]).reshape(2, NCH, K)
    dst = jnp.stack([edge_at[1], edge_ta[1]]).reshape(2, NCH, K)
    zrow = jnp.zeros((RPT, H), jnp.float32)
    zdeg = jnp.zeros((RPT,), jnp.float32)
    onesb = jnp.ones((K,), jnp.float32)

    hs = _embed(xs, w_e, b_e)
    sums, degs = _segsum_deg(hs, src, dst, zrow, zdeg, onesb)
    degs = degs[:, :, None]
    c1 = p["conv1"]
    hs = _conv(hs, sums, degs,
               jnp.stack([c1["Ws_a"], c1["Ws_t"]]),
               jnp.stack([c1["Wn_a"], c1["Wn_t"]]),
               jnp.stack([c1["b_a"], c1["b_t"]])[:, None, :])
    sums2 = _segsum_nodeg(hs, src, dst, zrow)

    c2 = p["conv2"]
    w_sym = jnp.stack([p["W_sym_a"], p["W_sym_t"]])
    b_sym = jnp.stack([p["b_sym_a"], p["b_sym_t"]])[:, None, :]
    wh = jnp.stack([
        jnp.concatenate([p["W_apv"], p["W_apc"], p["W_av"]], axis=1),
        jnp.concatenate([p["W_tpv"], p["W_tpc"], p["W_tv"]], axis=1),
    ])
    bh = jnp.stack([
        jnp.concatenate([p["b_apv"], p["b_apc"], p["b_av"]]),
        jnp.concatenate([p["b_tpv"], p["b_tpc"], p["b_tv"]]),
    ])[:, None, :]
    out = _conv_heads(hs, sums2, degs,
                      jnp.stack([c2["Ws_a"], c2["Ws_t"]]),
                      jnp.stack([c2["Wn_a"], c2["Wn_t"]]),
                      jnp.stack([c2["b_a"], c2["b_t"]])[:, None, :],
                      w_sym, b_sym, wh, bh)
    return (out[:NA, 0:1], out[:NA, 1:2], out[:NA, 2:3],
            out[NA:, 0:1], out[NA:, 1:2], out[NA:, 2:3])
